# baseline (speedup 1.0000x reference)
import functools

import jax
import jax.numpy as jnp
from jax import lax
from jax.experimental import pallas as pl
from jax.experimental.pallas import tpu as pltpu
from jax.experimental.pallas import tpu_sc as plsc

N_USERS = 25000
N_ITEMS = 25000
N_NODES = N_USERS + N_ITEMS
DIM = 64
N_LAYERS = 3
BATCH = 1024

NC = 2
NS = 16
NW = NC * NS
RPT = 1568
NPAD = NW * RPT
PC = 192
N_EDGES = 800000
NCHT = (N_EDGES + PC - 1) // PC
EPAD = (NCHT + 1) * PC
UPT = BATCH // NW


def _mesh():
    return plsc.VectorSubcoreMesh(core_axis_name="c", subcore_axis_name="s")


def _make_layer():
    @functools.partial(
        pl.kernel,
        mesh=_mesh(),
        compiler_params=pltpu.CompilerParams(use_tc_tiling_on_sc=False),
        out_type=jax.ShapeDtypeStruct((NPAD, DIM), jnp.float32),
        scratch_types=[
            pltpu.VMEM((96,), jnp.int32),
            pltpu.VMEM((3 * PC,), jnp.float32),
            pltpu.VMEM((3 * PC,), jnp.float32),
            pltpu.VMEM((PC,), jnp.int32),
            pltpu.VMEM((PC,), jnp.int32),
            pltpu.VMEM((PC, DIM), jnp.float32),
            pltpu.VMEM((PC, DIM), jnp.float32),
            pltpu.VMEM((RPT + 1, DIM), jnp.float32),
            pltpu.SemaphoreType.DMA,
        ],
    )
    def layer(emb_hbm, packed_hbm, cb_hbm, out_hbm,
              cb_v, buf0, buf1, si0, si1, rows0, rows1, acc_v, sem):
        wid = lax.axis_index("s") * NC + lax.axis_index("c")
        base = wid * RPT
        pltpu.sync_copy(cb_hbm, cb_v)
        c_lo = cb_v[pl.ds(wid, 16)][0]
        c_hi = cb_v[pl.ds(48 + wid, 16)][0]
        npair = (c_hi - c_lo + 1) // 2

        zv = jnp.zeros((16,), jnp.float32)

        def zero_body(i, carry):
            for j in range(DIM // 16):
                acc_v[i, pl.ds(j * 16, 16)] = zv
            return carry

        lax.fori_loop(0, RPT + 1, zero_body, 0)

        def fetch(buf, si, rows, c):
            pltpu.sync_copy(packed_hbm.at[c], buf)
            for g in range(PC // 16):
                s = pl.ds(g * 16, 16)
                si[s] = buf[pl.ds(PC + g * 16, 16)].astype(jnp.int32)
            pltpu.async_copy(emb_hbm.at[si.at[pl.ds(0, PC // 2)]],
                             rows.at[pl.ds(0, PC // 2)], sem)
            pltpu.async_copy(emb_hbm.at[si.at[pl.ds(PC // 2, PC // 2)]],
                             rows.at[pl.ds(PC // 2, PC // 2)], sem)

        def drain(rows):
            pltpu.make_async_copy(emb_hbm.at[pl.ds(0, PC)], rows, sem).wait()

        def process(buf, rows):
            def group_body(g, carry):
                eA = g * 16
                eB = PC // 2 + g * 16
                lds = []
                ws = []
                for e0 in (eA, eB):
                    ldv_ = buf[pl.ds(e0, 16)].astype(jnp.int32) - base
                    okv = (ldv_ >= 0) & (ldv_ < RPT)
                    lds.append(jnp.where(okv, ldv_, RPT))
                    ws.append(buf[pl.ds(2 * PC + e0, 16)])
                for i in range(16):
                    for h, e0 in enumerate((eA, eB)):
                        wt = ws[h][i]
                        ldc = lds[h][i]
                        for j in range(DIM // 16):
                            s = pl.ds(j * 16, 16)
                            plsc.addupdate(acc_v.at[ldc, s],
                                           rows[e0 + i, s] * wt)
                return carry

            lax.fori_loop(0, PC // 32, group_body, 0)

        @pl.when(npair > 0)
        def _():
            fetch(buf0, si0, rows0, c_lo)

        def pair_body(m, carry):
            k1 = 2 * m + 1
            fetch(buf1, si1, rows1, c_lo + k1)
            drain(rows0)
            process(buf0, rows0)

            @pl.when(k1 + 1 < 2 * npair)
            def _():
                fetch(buf0, si0, rows0, c_lo + k1 + 1)

            drain(rows1)
            process(buf1, rows1)
            return carry

        lax.fori_loop(0, npair, pair_body, 0)
        pltpu.sync_copy(acc_v.at[pl.ds(0, RPT)], out_hbm.at[pl.ds(base, RPT)])

    return layer


def _make_users_mean():
    @functools.partial(
        pl.kernel,
        mesh=_mesh(),
        compiler_params=pltpu.CompilerParams(use_tc_tiling_on_sc=False),
        out_type=jax.ShapeDtypeStruct((BATCH, DIM), jnp.float32),
        scratch_types=[
            pltpu.VMEM((UPT,), jnp.int32),
            pltpu.VMEM((UPT, DIM), jnp.float32),
            pltpu.VMEM((UPT, DIM), jnp.float32),
            pltpu.VMEM((UPT, DIM), jnp.float32),
            pltpu.VMEM((UPT, DIM), jnp.float32),
            pltpu.SemaphoreType.DMA,
        ],
    )
    def umean(e0_hbm, e1_hbm, e2_hbm, e3_hbm, users_hbm, out_hbm,
              uidx_v, r0, r1, r2, r3, sem):
        wid = lax.axis_index("s") * NC + lax.axis_index("c")
        ub = wid * UPT
        pltpu.sync_copy(users_hbm.at[pl.ds(ub, UPT)], uidx_v)
        cps = [
            pltpu.async_copy(e0_hbm.at[uidx_v], r0, sem),
            pltpu.async_copy(e1_hbm.at[uidx_v], r1, sem),
            pltpu.async_copy(e2_hbm.at[uidx_v], r2, sem),
            pltpu.async_copy(e3_hbm.at[uidx_v], r3, sem),
        ]
        for cp in cps:
            cp.wait()
        for r in range(UPT):
            for j in range(DIM // 16):
                s = pl.ds(j * 16, 16)
                r0[r, s] = (r0[r, s] + r1[r, s] + r2[r, s] + r3[r, s]) * 0.25
        pltpu.sync_copy(r0, out_hbm.at[pl.ds(ub, UPT)])

    return umean


IPAD = 25600
IB = 1280


def _rating(u, i0, i1, i2, i3):
    def body(u_ref, a_ref, b_ref, c_ref, d_ref, o_ref):
        it = (a_ref[...] + b_ref[...] + c_ref[...] + d_ref[...]) * 0.25
        acc = lax.dot_general(u_ref[...], it, (((1,), (1,)), ((), ())),
                              preferred_element_type=jnp.float32)
        o_ref[...] = jax.nn.sigmoid(acc)

    item_spec = pl.BlockSpec((IB, DIM), lambda i: (i, 0))
    out = pl.pallas_call(
        body,
        grid=(IPAD // IB,),
        in_specs=[
            pl.BlockSpec((BATCH, DIM), lambda i: (0, 0)),
            item_spec, item_spec, item_spec, item_spec,
        ],
        out_specs=pl.BlockSpec((BATCH, IB), lambda i: (0, i)),
        out_shape=jax.ShapeDtypeStruct((BATCH, IPAD), jnp.float32),
    )(u, i0, i1, i2, i3)
    return out[:, :N_ITEMS]


def kernel(user_emb, item_emb, edge_index, edge_weight, users):
    n_edges = edge_index.shape[1]
    src = edge_index[0].astype(jnp.int32)
    dst = edge_index[1].astype(jnp.int32)
    w = edge_weight.astype(jnp.float32)

    order = jnp.argsort(dst)
    src_s = src[order]
    dst_s = dst[order]
    w_s = w[order]

    bounds = jnp.arange(NW + 1, dtype=jnp.int32) * RPT
    starts = jnp.searchsorted(dst_s, bounds).astype(jnp.int32)
    c_lo = starts[:NW] // PC
    c_hi = (starts[1:] + PC - 1) // PC
    cb = jnp.concatenate([jnp.pad(c_lo, (0, 48 - NW)),
                          jnp.pad(c_hi, (0, 48 - NW))])

    pad_n = EPAD - n_edges
    dst_p = jnp.pad(dst_s, (0, pad_n)).astype(jnp.float32).reshape(-1, PC)
    src_p = jnp.pad(src_s, (0, pad_n)).astype(jnp.float32).reshape(-1, PC)
    w_p = jnp.pad(w_s, (0, pad_n)).reshape(-1, PC)
    packed = jnp.stack([dst_p, src_p, w_p], axis=1).reshape(-1, 3 * PC)

    emb0 = jnp.pad(jnp.concatenate([user_emb, item_emb], axis=0),
                   ((0, NPAD - N_NODES), (0, 0)))

    layer = _make_layer()
    e1 = layer(emb0, packed, cb)
    e2 = layer(e1, packed, cb)
    e3 = layer(e2, packed, cb)

    u = _make_users_mean()(emb0, e1, e2, e3, users.astype(jnp.int32))
    items = [jnp.pad(e[N_USERS:N_NODES], ((0, IPAD - N_ITEMS), (0, 0)))
             for e in (emb0, e1, e2, e3)]
    return _rating(u, *items)


# speedup vs baseline: 1.0117x; 1.0117x over previous
import functools

import jax
import jax.numpy as jnp
from jax import lax
from jax.experimental import pallas as pl
from jax.experimental.pallas import tpu as pltpu
from jax.experimental.pallas import tpu_sc as plsc

N_USERS = 25000
N_ITEMS = 25000
N_NODES = N_USERS + N_ITEMS
DIM = 64
N_LAYERS = 3
BATCH = 1024

NC = 2
NS = 16
NW = NC * NS
RPT = 1568
NPAD = NW * RPT
PC = 208
N_EDGES = 800000
NCHT = (N_EDGES + PC - 1) // PC
EPAD = (NCHT + 1) * PC
UPT = BATCH // NW


def _mesh():
    return plsc.VectorSubcoreMesh(core_axis_name="c", subcore_axis_name="s")


def _make_layer():
    @functools.partial(
        pl.kernel,
        mesh=_mesh(),
        compiler_params=pltpu.CompilerParams(use_tc_tiling_on_sc=False),
        out_type=jax.ShapeDtypeStruct((NPAD, DIM), jnp.float32),
        scratch_types=[
            pltpu.VMEM((96,), jnp.int32),
            pltpu.VMEM((3 * PC,), jnp.float32),
            pltpu.VMEM((3 * PC,), jnp.float32),
            pltpu.VMEM((PC,), jnp.int32),
            pltpu.VMEM((PC,), jnp.int32),
            pltpu.VMEM((PC, DIM), jnp.float32),
            pltpu.VMEM((PC, DIM), jnp.float32),
            pltpu.VMEM((RPT + 1, DIM), jnp.float32),
            pltpu.SemaphoreType.DMA,
        ],
    )
    def layer(emb_hbm, packed_hbm, cb_hbm, out_hbm,
              cb_v, buf0, buf1, si0, si1, rows0, rows1, acc_v, sem):
        wid = lax.axis_index("s") * NC + lax.axis_index("c")
        base = wid * RPT
        pltpu.sync_copy(cb_hbm, cb_v)
        c_lo = cb_v[pl.ds(wid, 16)][0]
        c_hi = cb_v[pl.ds(48 + wid, 16)][0]
        npair = (c_hi - c_lo + 1) // 2

        zv = jnp.zeros((16,), jnp.float32)

        def zero_body(i, carry):
            for j in range(DIM // 16):
                acc_v[i, pl.ds(j * 16, 16)] = zv
            return carry

        lax.fori_loop(0, RPT + 1, zero_body, 0)

        def fetch(buf, si, rows, c):
            pltpu.sync_copy(packed_hbm.at[c], buf)
            for g in range(PC // 16):
                s = pl.ds(g * 16, 16)
                si[s] = buf[pl.ds(PC + g * 16, 16)].astype(jnp.int32)
            pltpu.async_copy(emb_hbm.at[si.at[pl.ds(0, PC // 2)]],
                             rows.at[pl.ds(0, PC // 2)], sem)
            pltpu.async_copy(emb_hbm.at[si.at[pl.ds(PC // 2, PC // 2)]],
                             rows.at[pl.ds(PC // 2, PC // 2)], sem)

        def drain(rows):
            pltpu.make_async_copy(emb_hbm.at[pl.ds(0, PC)], rows, sem).wait()

        def process(buf, rows):
            def group_body(g, carry):
                e0 = g * 16
                ldv_ = buf[pl.ds(e0, 16)].astype(jnp.int32) - base
                okv = (ldv_ >= 0) & (ldv_ < RPT)
                ldv = jnp.where(okv, ldv_, RPT)
                wv = buf[pl.ds(2 * PC + e0, 16)]
                for i in range(16):
                    wt = wv[i]
                    ldc = ldv[i]
                    for j in range(DIM // 16):
                        s = pl.ds(j * 16, 16)
                        plsc.addupdate(acc_v.at[ldc, s], rows[e0 + i, s] * wt)
                return carry

            lax.fori_loop(0, PC // 16, group_body, 0)

        @pl.when(npair > 0)
        def _():
            fetch(buf0, si0, rows0, c_lo)

        def pair_body(m, carry):
            k1 = 2 * m + 1
            fetch(buf1, si1, rows1, c_lo + k1)
            drain(rows0)
            process(buf0, rows0)

            @pl.when(k1 + 1 < 2 * npair)
            def _():
                fetch(buf0, si0, rows0, c_lo + k1 + 1)

            drain(rows1)
            process(buf1, rows1)
            return carry

        lax.fori_loop(0, npair, pair_body, 0)
        pltpu.sync_copy(acc_v.at[pl.ds(0, RPT)], out_hbm.at[pl.ds(base, RPT)])

    return layer


def _make_users_mean():
    @functools.partial(
        pl.kernel,
        mesh=_mesh(),
        compiler_params=pltpu.CompilerParams(use_tc_tiling_on_sc=False),
        out_type=jax.ShapeDtypeStruct((BATCH, DIM), jnp.float32),
        scratch_types=[
            pltpu.VMEM((UPT,), jnp.int32),
            pltpu.VMEM((UPT, DIM), jnp.float32),
            pltpu.VMEM((UPT, DIM), jnp.float32),
            pltpu.VMEM((UPT, DIM), jnp.float32),
            pltpu.VMEM((UPT, DIM), jnp.float32),
            pltpu.SemaphoreType.DMA,
        ],
    )
    def umean(e0_hbm, e1_hbm, e2_hbm, e3_hbm, users_hbm, out_hbm,
              uidx_v, r0, r1, r2, r3, sem):
        wid = lax.axis_index("s") * NC + lax.axis_index("c")
        ub = wid * UPT
        pltpu.sync_copy(users_hbm.at[pl.ds(ub, UPT)], uidx_v)
        cps = [
            pltpu.async_copy(e0_hbm.at[uidx_v], r0, sem),
            pltpu.async_copy(e1_hbm.at[uidx_v], r1, sem),
            pltpu.async_copy(e2_hbm.at[uidx_v], r2, sem),
            pltpu.async_copy(e3_hbm.at[uidx_v], r3, sem),
        ]
        for cp in cps:
            cp.wait()
        for r in range(UPT):
            for j in range(DIM // 16):
                s = pl.ds(j * 16, 16)
                r0[r, s] = (r0[r, s] + r1[r, s] + r2[r, s] + r3[r, s]) * 0.25
        pltpu.sync_copy(r0, out_hbm.at[pl.ds(ub, UPT)])

    return umean


IPAD = 25600
IB = 1280


def _rating(u, i0, i1, i2, i3):
    def body(u_ref, a_ref, b_ref, c_ref, d_ref, o_ref):
        it = (a_ref[...] + b_ref[...] + c_ref[...] + d_ref[...]) * 0.25
        acc = lax.dot_general(u_ref[...], it, (((1,), (1,)), ((), ())),
                              preferred_element_type=jnp.float32)
        o_ref[...] = jax.nn.sigmoid(acc)

    item_spec = pl.BlockSpec((IB, DIM), lambda i: (i, 0))
    out = pl.pallas_call(
        body,
        grid=(IPAD // IB,),
        in_specs=[
            pl.BlockSpec((BATCH, DIM), lambda i: (0, 0)),
            item_spec, item_spec, item_spec, item_spec,
        ],
        out_specs=pl.BlockSpec((BATCH, IB), lambda i: (0, i)),
        out_shape=jax.ShapeDtypeStruct((BATCH, IPAD), jnp.float32),
    )(u, i0, i1, i2, i3)
    return out[:, :N_ITEMS]


def kernel(user_emb, item_emb, edge_index, edge_weight, users):
    n_edges = edge_index.shape[1]
    src = edge_index[0].astype(jnp.int32)
    dst = edge_index[1].astype(jnp.int32)
    w = edge_weight.astype(jnp.float32)

    order = jnp.argsort(dst)
    src_s = src[order]
    dst_s = dst[order]
    w_s = w[order]

    bounds = jnp.arange(NW + 1, dtype=jnp.int32) * RPT
    starts = jnp.searchsorted(dst_s, bounds).astype(jnp.int32)
    c_lo = starts[:NW] // PC
    c_hi = (starts[1:] + PC - 1) // PC
    cb = jnp.concatenate([jnp.pad(c_lo, (0, 48 - NW)),
                          jnp.pad(c_hi, (0, 48 - NW))])

    pad_n = EPAD - n_edges
    dst_p = jnp.pad(dst_s, (0, pad_n)).astype(jnp.float32).reshape(-1, PC)
    src_p = jnp.pad(src_s, (0, pad_n)).astype(jnp.float32).reshape(-1, PC)
    w_p = jnp.pad(w_s, (0, pad_n)).reshape(-1, PC)
    packed = jnp.stack([dst_p, src_p, w_p], axis=1).reshape(-1, 3 * PC)

    emb0 = jnp.pad(jnp.concatenate([user_emb, item_emb], axis=0),
                   ((0, NPAD - N_NODES), (0, 0)))

    layer = _make_layer()
    e1 = layer(emb0, packed, cb)
    e2 = layer(e1, packed, cb)
    e3 = layer(e2, packed, cb)

    u = _make_users_mean()(emb0, e1, e2, e3, users.astype(jnp.int32))
    items = [jnp.pad(e[N_USERS:N_NODES], ((0, IPAD - N_ITEMS), (0, 0)))
             for e in (emb0, e1, e2, e3)]
    return _rating(u, *items)
